# baseline (speedup 1.0000x reference)
import functools

import jax
import jax.numpy as jnp
from jax import lax
from jax.experimental import pallas as pl
from jax.experimental.pallas import tpu as pltpu
from jax.experimental.pallas import tpu_sc as plsc

V_TILE = 2048
V_TILE1 = 4096
NEG = -1e30


def _gather_sc(emb, idx_flat):
    B = idx_flat.shape[0]
    D = emb.shape[1]
    info = plsc.get_sparse_core_info()
    NC, NS = info.num_cores, info.num_subcores
    NW = NC * NS
    b_per_w = B // NW
    mesh = plsc.VectorSubcoreMesh(core_axis_name="c", subcore_axis_name="s")

    @functools.partial(
        pl.kernel,
        mesh=mesh,
        compiler_params=pltpu.CompilerParams(use_tc_tiling_on_sc=False),
        out_type=jax.ShapeDtypeStruct((B, D), jnp.float32),
        scratch_types=[
            pltpu.VMEM((b_per_w,), jnp.int32),
            pltpu.VMEM((b_per_w, D), jnp.float32),
            pltpu.SemaphoreType.DMA,
        ],
    )
    def k(table_hbm, idx_hbm, out_hbm, idx_v, rows_v, sem):
        wid = lax.axis_index("s") * NC + lax.axis_index("c")
        base = wid * b_per_w
        pltpu.sync_copy(idx_hbm.at[pl.ds(base, b_per_w)], idx_v)
        pltpu.async_copy(table_hbm.at[idx_v], rows_v, sem).wait()
        pltpu.sync_copy(rows_v, out_hbm.at[pl.ds(base, b_per_w)])

    return k(emb, idx_flat)


def _hid_body(concat_ref, w1_ref, b1_ref, hid_ref):
    h = lax.dot_general(concat_ref[...], w1_ref[...],
                        (((1,), (1,)), ((), ())),
                        preferred_element_type=jnp.float32)
    hid_ref[...] = jnp.maximum(h + b1_ref[...], 0.0).astype(jnp.bfloat16)


def _p1_body(nv, vocab, hid_ref, w2_ref, b2_ref, tmax_ref, ssum_ref):
    j = pl.program_id(0)
    logits = lax.dot_general(hid_ref[...], w2_ref[...],
                             (((1,), (1,)), ((), ())),
                             preferred_element_type=jnp.float32) + b2_ref[...]

    @pl.when(j < nv - 1)
    def _():
        t = jnp.max(logits, axis=1, keepdims=True)
        s = jnp.sum(jnp.exp(logits - t), axis=1, keepdims=True)
        tmax_ref[...] = jnp.broadcast_to(t, tmax_ref.shape)
        ssum_ref[...] = jnp.broadcast_to(s, ssum_ref.shape)

    @pl.when(j == nv - 1)
    def _():
        col = j * V_TILE1 + lax.broadcasted_iota(jnp.int32, logits.shape, 1)
        lm = jnp.where(col < vocab, logits, NEG)
        t = jnp.max(lm, axis=1, keepdims=True)
        s = jnp.sum(jnp.where(col < vocab, jnp.exp(lm - t), 0.0),
                    axis=1, keepdims=True)
        tmax_ref[...] = jnp.broadcast_to(t, tmax_ref.shape)
        ssum_ref[...] = jnp.broadcast_to(s, ssum_ref.shape)


def _comb_body(tmax_ref, ssum_ref, lse_ref):
    m = jnp.max(tmax_ref[...], axis=1, keepdims=True)
    s = jnp.sum(ssum_ref[...] * jnp.exp(tmax_ref[...] - m),
                axis=1, keepdims=True)
    lse_ref[...] = m + jnp.log(s) - jnp.log(jnp.float32(128.0))


def _p2_body(hid_ref, w2_ref, b2_ref, lse_ref, out_ref):
    logits = lax.dot_general(hid_ref[...], w2_ref[...],
                             (((1,), (1,)), ((), ())),
                             preferred_element_type=jnp.float32) + b2_ref[...]
    out_ref[...] = logits - lse_ref[...]


def kernel(inputs, emb, W1, b1, W2, b2):
    batch = inputs.shape[0]
    vocab, hidden = W2.shape
    in_dim = W1.shape[1]
    nv = pl.cdiv(vocab, V_TILE)

    concat = _gather_sc(emb, inputs.reshape(-1)).reshape(batch, in_dim)
    b1r = b1.reshape(1, -1)
    b2r = b2.reshape(1, -1)
    W2b = W2.astype(jnp.bfloat16)

    hid = pl.pallas_call(
        _hid_body,
        in_specs=[
            pl.BlockSpec((batch, in_dim), lambda: (0, 0)),
            pl.BlockSpec((hidden, in_dim), lambda: (0, 0)),
            pl.BlockSpec((1, hidden), lambda: (0, 0)),
        ],
        out_specs=pl.BlockSpec((batch, hidden), lambda: (0, 0)),
        out_shape=jax.ShapeDtypeStruct((batch, hidden), jnp.bfloat16),
    )(concat, W1, b1r)

    nv1 = pl.cdiv(vocab, V_TILE1)
    tmax, ssum = pl.pallas_call(
        functools.partial(_p1_body, nv1, vocab),
        grid=(nv1,),
        in_specs=[
            pl.BlockSpec((batch, hidden), lambda j: (0, 0)),
            pl.BlockSpec((V_TILE1, hidden), lambda j: (j, 0)),
            pl.BlockSpec((1, V_TILE1), lambda j: (0, j)),
        ],
        out_specs=[
            pl.BlockSpec((batch, 128), lambda j: (0, j)),
            pl.BlockSpec((batch, 128), lambda j: (0, j)),
        ],
        out_shape=[
            jax.ShapeDtypeStruct((batch, nv1 * 128), jnp.float32),
            jax.ShapeDtypeStruct((batch, nv1 * 128), jnp.float32),
        ],
        compiler_params=pltpu.CompilerParams(
            dimension_semantics=("parallel",)),
    )(hid, W2b, b2r)

    lse = pl.pallas_call(
        _comb_body,
        in_specs=[
            pl.BlockSpec((batch, nv1 * 128), lambda: (0, 0)),
            pl.BlockSpec((batch, nv1 * 128), lambda: (0, 0)),
        ],
        out_specs=pl.BlockSpec((batch, 1), lambda: (0, 0)),
        out_shape=jax.ShapeDtypeStruct((batch, 1), jnp.float32),
    )(tmax, ssum)

    out = pl.pallas_call(
        _p2_body,
        grid=(nv,),
        in_specs=[
            pl.BlockSpec((batch, hidden), lambda j: (0, 0)),
            pl.BlockSpec((V_TILE, hidden), lambda j: (j, 0)),
            pl.BlockSpec((1, V_TILE), lambda j: (0, j)),
            pl.BlockSpec((batch, 1), lambda j: (0, 0)),
        ],
        out_specs=pl.BlockSpec((batch, V_TILE), lambda j: (0, j)),
        out_shape=jax.ShapeDtypeStruct((batch, vocab), jnp.float32),
        compiler_params=pltpu.CompilerParams(
            dimension_semantics=("parallel",)),
    )(hid, W2b, b2r, lse)

    return out


# speedup vs baseline: 1.0315x; 1.0315x over previous
import functools

import jax
import jax.numpy as jnp
from jax import lax
from jax.experimental import pallas as pl
from jax.experimental.pallas import tpu as pltpu
from jax.experimental.pallas import tpu_sc as plsc

V_TILE = 2048
V_TILE1 = 4096
NEG = -1e30


def _gather_sc(emb, idx_flat):
    B = idx_flat.shape[0]
    D = emb.shape[1]
    info = plsc.get_sparse_core_info()
    NC, NS = info.num_cores, info.num_subcores
    NW = NC * NS
    b_per_w = B // NW
    mesh = plsc.VectorSubcoreMesh(core_axis_name="c", subcore_axis_name="s")

    @functools.partial(
        pl.kernel,
        mesh=mesh,
        compiler_params=pltpu.CompilerParams(use_tc_tiling_on_sc=False),
        out_type=jax.ShapeDtypeStruct((B, D), jnp.float32),
        scratch_types=[
            pltpu.VMEM((b_per_w,), jnp.int32),
            pltpu.VMEM((b_per_w, D), jnp.float32),
            pltpu.SemaphoreType.DMA,
        ],
    )
    def k(table_hbm, idx_hbm, out_hbm, idx_v, rows_v, sem):
        wid = lax.axis_index("s") * NC + lax.axis_index("c")
        base = wid * b_per_w
        pltpu.sync_copy(idx_hbm.at[pl.ds(base, b_per_w)], idx_v)
        pltpu.async_copy(table_hbm.at[idx_v], rows_v, sem).wait()
        pltpu.sync_copy(rows_v, out_hbm.at[pl.ds(base, b_per_w)])

    return k(emb, idx_flat)


def _hid_body(concat_ref, w1_ref, b1_ref, hid_ref):
    h = lax.dot_general(concat_ref[...], w1_ref[...],
                        (((1,), (1,)), ((), ())),
                        preferred_element_type=jnp.float32)
    hid_ref[...] = jnp.maximum(h + b1_ref[...], 0.0).astype(jnp.bfloat16)


def _p1_body(nv, vocab, hid_ref, w2_ref, b2_ref, tmax_ref, ssum_ref):
    j = pl.program_id(0)
    logits = lax.dot_general(hid_ref[...], w2_ref[...].astype(jnp.bfloat16),
                             (((1,), (1,)), ((), ())),
                             preferred_element_type=jnp.float32) + b2_ref[...]

    @pl.when(j < nv - 1)
    def _():
        t = jnp.max(logits, axis=1, keepdims=True)
        s = jnp.sum(jnp.exp(logits - t), axis=1, keepdims=True)
        tmax_ref[...] = jnp.broadcast_to(t, tmax_ref.shape)
        ssum_ref[...] = jnp.broadcast_to(s, ssum_ref.shape)

    @pl.when(j == nv - 1)
    def _():
        col = j * V_TILE1 + lax.broadcasted_iota(jnp.int32, logits.shape, 1)
        lm = jnp.where(col < vocab, logits, NEG)
        t = jnp.max(lm, axis=1, keepdims=True)
        s = jnp.sum(jnp.where(col < vocab, jnp.exp(lm - t), 0.0),
                    axis=1, keepdims=True)
        tmax_ref[...] = jnp.broadcast_to(t, tmax_ref.shape)
        ssum_ref[...] = jnp.broadcast_to(s, ssum_ref.shape)


def _comb_body(tmax_ref, ssum_ref, lse_ref):
    m = jnp.max(tmax_ref[...], axis=1, keepdims=True)
    s = jnp.sum(ssum_ref[...] * jnp.exp(tmax_ref[...] - m),
                axis=1, keepdims=True)
    lse_ref[...] = m + jnp.log(s) - jnp.log(jnp.float32(128.0))


def _p2_body(hid_ref, w2_ref, b2_ref, lse_ref, out_ref):
    logits = lax.dot_general(hid_ref[...], w2_ref[...].astype(jnp.bfloat16),
                             (((1,), (1,)), ((), ())),
                             preferred_element_type=jnp.float32) + b2_ref[...]
    out_ref[...] = logits - lse_ref[...]


def kernel(inputs, emb, W1, b1, W2, b2):
    batch = inputs.shape[0]
    vocab, hidden = W2.shape
    in_dim = W1.shape[1]
    nv = pl.cdiv(vocab, V_TILE)

    concat = _gather_sc(emb, inputs.reshape(-1)).reshape(batch, in_dim)
    b1r = b1.reshape(1, -1)
    b2r = b2.reshape(1, -1)

    hid = pl.pallas_call(
        _hid_body,
        in_specs=[
            pl.BlockSpec((batch, in_dim), lambda: (0, 0)),
            pl.BlockSpec((hidden, in_dim), lambda: (0, 0)),
            pl.BlockSpec((1, hidden), lambda: (0, 0)),
        ],
        out_specs=pl.BlockSpec((batch, hidden), lambda: (0, 0)),
        out_shape=jax.ShapeDtypeStruct((batch, hidden), jnp.bfloat16),
    )(concat, W1, b1r)

    nv1 = pl.cdiv(vocab, V_TILE1)
    tmax, ssum = pl.pallas_call(
        functools.partial(_p1_body, nv1, vocab),
        grid=(nv1,),
        in_specs=[
            pl.BlockSpec((batch, hidden), lambda j: (0, 0)),
            pl.BlockSpec((V_TILE1, hidden), lambda j: (j, 0)),
            pl.BlockSpec((1, V_TILE1), lambda j: (0, j)),
        ],
        out_specs=[
            pl.BlockSpec((batch, 128), lambda j: (0, j)),
            pl.BlockSpec((batch, 128), lambda j: (0, j)),
        ],
        out_shape=[
            jax.ShapeDtypeStruct((batch, nv1 * 128), jnp.float32),
            jax.ShapeDtypeStruct((batch, nv1 * 128), jnp.float32),
        ],
        compiler_params=pltpu.CompilerParams(
            dimension_semantics=("parallel",)),
    )(hid, W2, b2r)

    lse = pl.pallas_call(
        _comb_body,
        in_specs=[
            pl.BlockSpec((batch, nv1 * 128), lambda: (0, 0)),
            pl.BlockSpec((batch, nv1 * 128), lambda: (0, 0)),
        ],
        out_specs=pl.BlockSpec((batch, 1), lambda: (0, 0)),
        out_shape=jax.ShapeDtypeStruct((batch, 1), jnp.float32),
    )(tmax, ssum)

    out = pl.pallas_call(
        _p2_body,
        grid=(nv,),
        in_specs=[
            pl.BlockSpec((batch, hidden), lambda j: (0, 0)),
            pl.BlockSpec((V_TILE, hidden), lambda j: (j, 0)),
            pl.BlockSpec((1, V_TILE), lambda j: (0, j)),
            pl.BlockSpec((batch, 1), lambda j: (0, 0)),
        ],
        out_specs=pl.BlockSpec((batch, V_TILE), lambda j: (0, j)),
        out_shape=jax.ShapeDtypeStruct((batch, vocab), jnp.float32),
        compiler_params=pltpu.CompilerParams(
            dimension_semantics=("parallel",)),
    )(hid, W2, b2r, lse)

    return out
